# Initial kernel scaffold; baseline (speedup 1.0000x reference)
import functools

import jax
import jax.numpy as jnp
from jax import lax
from jax.experimental import pallas as pl
from jax.experimental.pallas import tpu as pltpu

NUM_EXPERTS = 64
TOP_K = 8
BLOCK_T = 512


def _router_block(x_ref, w_ref, wout_ref, iout_ref):
    x = x_ref[...]
    w = w_ref[...]
    logits = lax.dot_general(
        x, w, (((1,), (1,)), ((), ())), preferred_element_type=jnp.float32
    )
    bt = logits.shape[0]
    e = logits.shape[1]
    iota_e = lax.broadcasted_iota(jnp.int32, (bt, e), 1)
    neg_inf = jnp.float32(-jnp.inf)

    work = logits
    mask = jnp.zeros((bt, e), dtype=jnp.bool_)
    idx_cols = []
    m1 = None
    for j in range(TOP_K):
        m = jnp.max(work, axis=1, keepdims=True)
        if j == 0:
            m1 = m
        is_m = work == m
        idx = jnp.min(jnp.where(is_m, iota_e, e), axis=1, keepdims=True)
        idx_cols.append(idx)
        taken = iota_e == idx
        mask = jnp.logical_or(mask, taken)
        work = jnp.where(taken, neg_inf, work)

    unnorm = jnp.where(mask, jnp.exp(logits - m1), jnp.float32(0.0))
    denom = jnp.sum(unnorm, axis=1, keepdims=True)
    wout_ref[...] = unnorm / denom
    iout_ref[...] = jnp.concatenate(idx_cols, axis=1)


@jax.jit
def kernel(input, W):
    b, s, d = input.shape
    e = W.shape[0]
    t = b * s
    x2 = input.reshape(t, d)
    bt = BLOCK_T if t % BLOCK_T == 0 else t
    grid = (t // bt,)

    weights, idx = pl.pallas_call(
        _router_block,
        grid=grid,
        in_specs=[
            pl.BlockSpec((bt, d), lambda i: (i, 0)),
            pl.BlockSpec((e, d), lambda i: (0, 0)),
        ],
        out_specs=[
            pl.BlockSpec((bt, e), lambda i: (i, 0)),
            pl.BlockSpec((bt, TOP_K), lambda i: (i, 0)),
        ],
        out_shape=[
            jax.ShapeDtypeStruct((t, e), jnp.float32),
            jax.ShapeDtypeStruct((t, TOP_K), jnp.int32),
        ],
    )(x2, W)

    return weights.reshape(b, s, e), idx.reshape(b, s, TOP_K)


# speedup vs baseline: 1.1531x; 1.1531x over previous
import jax
import jax.numpy as jnp
from jax import lax
from jax.experimental import pallas as pl
from jax.experimental.pallas import tpu as pltpu
from jax.experimental.pallas import tpu_sc as plsc

NUM_EXPERTS = 64
TOP_K = 8
BT = 512
NW = 32
L = 16


def _logits_block(x_ref, w_ref, out_ref):
    x = x_ref[...]
    w = w_ref[...]
    lt = lax.dot_general(
        w, x, (((1,), (1,)), ((), ())), preferred_element_type=jnp.float32
    )
    out_ref[...] = lt.reshape(1, 1, NUM_EXPERTS * BT)


def _sc_body(lin_hbm, wout_hbm, iout_hbm, lvm, wvm, ivm):
    nblk = lin_hbm.shape[0]
    blocks_per_w = nblk // NW
    cid = lax.axis_index("c")
    sid = lax.axis_index("s")
    wid = sid * 2 + cid
    lane = lax.broadcasted_iota(jnp.int32, (L,), 0)
    neg_inf = jnp.full((L,), -jnp.inf, dtype=jnp.float32)
    zeros = jnp.zeros((L,), jnp.float32)

    def do_block(c, _):
        blk = wid * blocks_per_w + c
        pltpu.sync_copy(lin_hbm.at[blk, 0], lvm)

        def zrow(r, _):
            for u in range(8):
                wvm[pl.ds((r * 8 + u) * L, L)] = zeros
            return _
        lax.fori_loop(0, BT * NUM_EXPERTS // (8 * L), zrow, 0)

        def group(g, _):
            base = g * L
            toks = base + lane
            vals = []
            idxs = []
            for j in range(TOP_K):
                m = neg_inf
                idx = jnp.zeros((L,), jnp.int32)
                for e in range(NUM_EXPERTS):
                    v = lvm[pl.ds(e * BT + base, L)]
                    upd = v > m
                    m = jnp.where(upd, v, m)
                    idx = jnp.where(upd, jnp.int32(e), idx)
                vals.append(m)
                idxs.append(idx)
                if j + 1 < TOP_K:
                    plsc.store_scatter(lvm, [idx * BT + toks], neg_inf)
            ws = [jnp.exp(v - vals[0]) for v in vals]
            s = ws[0]
            for j in range(1, TOP_K):
                s = s + ws[j]
            for j in range(TOP_K):
                plsc.store_scatter(
                    wvm, [toks * NUM_EXPERTS + idxs[j]], ws[j] / s
                )
                plsc.store_scatter(
                    ivm, [toks * TOP_K + j], idxs[j]
                )
            return _
        lax.fori_loop(0, BT // L, group, 0)

        pltpu.sync_copy(wvm, wout_hbm.at[pl.ds(blk * BT * NUM_EXPERTS,
                                               BT * NUM_EXPERTS)])
        pltpu.sync_copy(ivm, iout_hbm.at[pl.ds(blk * BT * TOP_K,
                                               BT * TOP_K)])
        return 0

    lax.fori_loop(0, blocks_per_w, do_block, 0)


@jax.jit
def kernel(input, W):
    b, s, d = input.shape
    e = W.shape[0]
    t = b * s
    x2 = input.reshape(t, d)
    nblk = t // BT

    logits_blocked = pl.pallas_call(
        _logits_block,
        grid=(nblk,),
        in_specs=[
            pl.BlockSpec((BT, d), lambda i: (i, 0)),
            pl.BlockSpec((e, d), lambda i: (0, 0)),
        ],
        out_specs=pl.BlockSpec((1, 1, e * BT), lambda i: (i, 0, 0)),
        out_shape=jax.ShapeDtypeStruct((nblk, 1, e * BT), jnp.float32),
    )(x2, W)

    mesh = plsc.VectorSubcoreMesh(core_axis_name="c", subcore_axis_name="s")
    weights, idx = pl.kernel(
        _sc_body,
        out_type=[
            jax.ShapeDtypeStruct((t * e,), jnp.float32),
            jax.ShapeDtypeStruct((t * TOP_K,), jnp.int32),
        ],
        mesh=mesh,
        compiler_params=pltpu.CompilerParams(needs_layout_passes=False),
        scratch_types=[
            pltpu.VMEM((e * BT,), jnp.float32),
            pltpu.VMEM((BT * e,), jnp.float32),
            pltpu.VMEM((BT * TOP_K,), jnp.int32),
        ],
    )(logits_blocked)

    return weights.reshape(b, s, e), idx.reshape(b, s, TOP_K)
